# Initial kernel scaffold; baseline (speedup 1.0000x reference)
import jax
import jax.numpy as jnp
from jax.experimental import pallas as pl


def kernel(x, edge_index, W, b):
    raise NotImplementedError("write your pallas kernel here")


# speedup vs baseline: 19.7494x; 19.7494x over previous
import functools

import jax
import jax.numpy as jnp
from jax import lax
from jax.experimental import pallas as pl
from jax.experimental.pallas import tpu as pltpu
from jax.experimental.pallas import tpu_sc as plsc

N = 10000
D = 128
E = 320000
NW = 32
NSUB = 16
EPW = E // NW
CH = 128
NCH = (EPW + CH - 1) // CH
EPAD = NCH * CH
NP = 10112
RPT = NP // NSUB
DEGL = 16

_mesh_args = dict(core_axis_name="c", subcore_axis_name="s", num_cores=2,
                  num_subcores=NSUB)


@functools.lru_cache(maxsize=None)
def _build_sc_kernels():
    mesh = plsc.VectorSubcoreMesh(**_mesh_args)

    @functools.partial(
        pl.kernel,
        out_type=jax.ShapeDtypeStruct((2, NP, DEGL), jnp.float32),
        mesh=mesh,
        compiler_params=pltpu.CompilerParams(use_tc_tiling_on_sc=False),
        scratch_types=[
            pltpu.VMEM((NCH, CH), jnp.int32),
            pltpu.VMEM((CH, DEGL), jnp.float32),
            pltpu.VMEM_SHARED((NP, DEGL), jnp.float32),
        ],
    )
    def deg_kernel(row_hbm, zeros_hbm, ones_hbm, deg_out, idx_v, ones_v,
                   deg_sp):
        c = lax.axis_index("c")
        s = lax.axis_index("s")
        w = c * NSUB + s
        pltpu.sync_copy(zeros_hbm.at[pl.ds(s * RPT, RPT)],
                        deg_sp.at[pl.ds(s * RPT, RPT)])
        pltpu.sync_copy(ones_hbm, ones_v)
        pltpu.sync_copy(row_hbm.at[w], idx_v)
        plsc.subcore_barrier()

        def body(j, carry):
            pltpu.sync_copy(ones_v, deg_sp.at[idx_v.at[j]], add=True)
            return carry

        lax.fori_loop(0, NCH, body, 0)
        plsc.subcore_barrier()
        pltpu.sync_copy(deg_sp.at[pl.ds(s * RPT, RPT)],
                        deg_out.at[c, pl.ds(s * RPT, RPT)])

    @functools.partial(
        pl.kernel,
        out_type=jax.ShapeDtypeStruct((2, NP, D), jnp.float32),
        mesh=mesh,
        scratch_types=[
            pltpu.VMEM((NCH, CH), jnp.int32),
            pltpu.VMEM((NCH, CH), jnp.int32),
            pltpu.VMEM((CH, D), jnp.float32),
            pltpu.VMEM_SHARED((NP, D), jnp.float32),
        ],
    )
    def agg_kernel(col_hbm, row_hbm, g_hbm, acc_out, col_v, row_v, buf_v,
                   acc_sp):
        c = lax.axis_index("c")
        s = lax.axis_index("s")
        w = c * NSUB + s
        pltpu.sync_copy(g_hbm.at[pl.ds(s * RPT, RPT)],
                        acc_sp.at[pl.ds(s * RPT, RPT)])
        pltpu.sync_copy(col_hbm.at[w], col_v)
        pltpu.sync_copy(row_hbm.at[w], row_v)
        plsc.subcore_barrier()

        def body(j, carry):
            pltpu.sync_copy(g_hbm.at[col_v.at[j]], buf_v)
            pltpu.sync_copy(buf_v, acc_sp.at[row_v.at[j]], add=True)
            return carry

        lax.fori_loop(0, NCH, body, 0)
        plsc.subcore_barrier()
        pltpu.sync_copy(acc_sp.at[pl.ds(s * RPT, RPT)],
                        acc_out.at[c, pl.ds(s * RPT, RPT)])

    return deg_kernel, agg_kernel


_RB = RPT


def _g_body(x_ref, w_ref, b_ref, deg_ref, g_ref):
    h = jnp.dot(x_ref[...], w_ref[...], preferred_element_type=jnp.float32)
    h = h + b_ref[...]
    deg = deg_ref[0, :, 0] + deg_ref[1, :, 0] + 1.0
    dis = lax.rsqrt(deg + 1e-12)
    g_ref[...] = h * dis[:, None]


def _out_body(acc_ref, g_ref, deg_ref, out_ref):
    deg = deg_ref[0, :, 0] + deg_ref[1, :, 0] + 1.0
    dis = lax.rsqrt(deg + 1e-12)
    s = acc_ref[0] + acc_ref[1] - g_ref[...]
    out_ref[...] = s * dis[:, None]


def kernel(x, edge_index, W, b):
    row = edge_index[0].reshape(NW, EPW)
    col = edge_index[1].reshape(NW, EPW)
    pad_rows = jnp.broadcast_to(
        jnp.arange(N, N + EPAD - EPW, dtype=row.dtype)[None], (NW, EPAD - EPW))
    row_p = jnp.concatenate([row, pad_rows], axis=1)
    col_p = jnp.pad(col, ((0, 0), (0, EPAD - EPW)), constant_values=0)
    row_p = row_p.reshape(NW, NCH, CH)
    col_p = col_p.reshape(NW, NCH, CH)
    zeros = jnp.zeros((NP, DEGL), jnp.float32)
    ones = jnp.ones((CH, DEGL), jnp.float32)
    b2 = b.reshape(1, D)

    deg_kernel, agg_kernel = _build_sc_kernels()
    deg = deg_kernel(row_p, zeros, ones)

    g = pl.pallas_call(
        _g_body,
        grid=(NP // _RB,),
        in_specs=[
            pl.BlockSpec((_RB, D), lambda i: (i, 0)),
            pl.BlockSpec((D, D), lambda i: (0, 0)),
            pl.BlockSpec((1, D), lambda i: (0, 0)),
            pl.BlockSpec((2, _RB, DEGL), lambda i: (0, i, 0)),
        ],
        out_specs=pl.BlockSpec((_RB, D), lambda i: (i, 0)),
        out_shape=jax.ShapeDtypeStruct((NP, D), jnp.float32),
    )(x, W, b2, deg)

    acc = agg_kernel(col_p, row_p, g)

    out = pl.pallas_call(
        _out_body,
        grid=(NP // _RB,),
        in_specs=[
            pl.BlockSpec((2, _RB, D), lambda i: (0, i, 0)),
            pl.BlockSpec((_RB, D), lambda i: (i, 0)),
            pl.BlockSpec((2, _RB, DEGL), lambda i: (0, i, 0)),
        ],
        out_specs=pl.BlockSpec((_RB, D), lambda i: (i, 0)),
        out_shape=jax.ShapeDtypeStruct((N, D), jnp.float32),
    )(acc, g, deg)
    return out
